# baseline (speedup 1.0000x reference)
import functools

import jax
import jax.numpy as jnp
from jax import lax
from jax.experimental import pallas as pl
from jax.experimental.pallas import tpu as pltpu
from jax.experimental.pallas import tpu_sc as plsc

VOCAB = 1000000
DIM = 64
BATCH = 16384
NEG = 20
INTERCEPT = 1.1

PDIM = 128
NC = 2
NS = 16
NW = NC * NS
BPW = BATCH // NW
C = 32
NCHUNK = BPW // C
NEG_PER_CHUNK = C * NEG
NIDX_COLS = 128
NEG_GATHERS = NEG_PER_CHUNK // NIDX_COLS
NVREG = DIM // 16


def _sc_dots(center_idx, context_idx, neg_idx, center_flat, context_emb, ws):
    mesh = plsc.VectorSubcoreMesh(core_axis_name="c", subcore_axis_name="s")

    @functools.partial(
        pl.kernel,
        mesh=mesh,
        out_type=[jax.ShapeDtypeStruct((BATCH * 16,), jnp.float32)] * 3,
        scratch_types=[
            pltpu.VMEM((C,), jnp.int32),
            pltpu.VMEM((C,), jnp.int32),
            pltpu.VMEM((NEG_PER_CHUNK,), jnp.int32),
            pltpu.VMEM((C * DIM,), jnp.int32),
            pltpu.VMEM((C * DIM,), jnp.float32),
            pltpu.VMEM((C, PDIM), jnp.float32),
            pltpu.VMEM((NEG_PER_CHUNK, PDIM), jnp.float32),
            pltpu.VMEM((C * 16,), jnp.float32),
            pltpu.VMEM((C * 16,), jnp.float32),
            pltpu.VMEM((C * 16,), jnp.float32),
            pltpu.VMEM((DIM,), jnp.float32),
            pltpu.SemaphoreType.DMA,
        ],
    )
    def k(cidx_hbm, uidx_hbm, nidx_hbm, cflat_hbm, uemb_hbm, ws_hbm,
          pos_out, neg_out, sent_out,
          cidx_v, uidx_v, nidx_v, cidx2_v, cvals, urows, nrows,
          posb, negb, sentb, ws_v, sem):
        wid = lax.axis_index("s") * NC + lax.axis_index("c")
        pltpu.sync_copy(ws_hbm, ws_v)
        wsv = [ws_v[pl.ds(kk * 16, 16)] for kk in range(NVREG)]
        dv = [(lax.iota(jnp.int32, 16) + 16 * kk) * VOCAB
              for kk in range(NVREG)]
        gdn = lax.GatherDimensionNumbers(
            offset_dims=(), collapsed_slice_dims=(0,), start_index_map=(0,))

        def bcast_lane(vec, r):
            rvec = jnp.full((16,), r, jnp.int32)
            return lax.gather(vec, rvec[:, None], dimension_numbers=gdn,
                              slice_sizes=(1,),
                              mode=lax.GatherScatterMode.PROMISE_IN_BOUNDS)

        def chunk_body(c, _):
            base = wid * BPW + c * C
            pltpu.sync_copy(cidx_hbm.at[pl.ds(base, C)], cidx_v)
            pltpu.sync_copy(uidx_hbm.at[pl.ds(base, C)], uidx_v)
            pltpu.sync_copy(nidx_hbm.at[pl.ds(base * NEG, NEG_PER_CHUNK)],
                            nidx_v)
            cp2 = pltpu.async_copy(uemb_hbm.at[uidx_v], urows, sem)
            cps = [
                pltpu.async_copy(
                    uemb_hbm.at[nidx_v.at[pl.ds(j * NIDX_COLS, NIDX_COLS)]],
                    nrows.at[pl.ds(j * NIDX_COLS, NIDX_COLS)], sem)
                for j in range(NEG_GATHERS)
            ]

            def cidx_body(b, _):
                civ = cidx_v[pl.ds((b // 16) * 16, 16)]
                svec = bcast_lane(civ, b % 16)
                for kk in range(NVREG):
                    cidx2_v[pl.ds(b * DIM + kk * 16, 16)] = dv[kk] + svec
                return 0

            lax.fori_loop(0, C, cidx_body, 0)
            ccps = [
                pltpu.async_copy(
                    cflat_hbm.at[cidx2_v.at[pl.ds(j * NIDX_COLS, NIDX_COLS)]],
                    cvals.at[pl.ds(j * NIDX_COLS, NIDX_COLS)], sem)
                for j in range(C * DIM // NIDX_COLS)
            ]
            cp2.wait()
            for cp in cps:
                cp.wait()
            for cp in ccps:
                cp.wait()

            def b_body(b, _):
                v = [cvals[pl.ds(b * DIM + kk * 16, 16)]
                     for kk in range(NVREG)]
                u = [urows[b, pl.ds(kk * 16, 16)] for kk in range(NVREG)]
                pos = v[0] * u[0]
                for kk in range(1, NVREG):
                    pos = pos + v[kk] * u[kk]
                sent = v[0] * wsv[0]
                for kk in range(1, NVREG):
                    sent = sent + v[kk] * wsv[kk]
                nacc = [nrows[b * NEG, pl.ds(kk * 16, 16)]
                        for kk in range(NVREG)]
                for n in range(1, NEG):
                    for kk in range(NVREG):
                        nacc[kk] = nacc[kk] + nrows[b * NEG + n,
                                                    pl.ds(kk * 16, 16)]
                neg = v[0] * nacc[0]
                for kk in range(1, NVREG):
                    neg = neg + v[kk] * nacc[kk]
                posb[pl.ds(b * 16, 16)] = pos
                sentb[pl.ds(b * 16, 16)] = sent
                negb[pl.ds(b * 16, 16)] = neg
                return 0

            lax.fori_loop(0, C, b_body, 0)
            pltpu.sync_copy(posb, pos_out.at[pl.ds(base * 16, C * 16)])
            pltpu.sync_copy(negb, neg_out.at[pl.ds(base * 16, C * 16)])
            pltpu.sync_copy(sentb, sent_out.at[pl.ds(base * 16, C * 16)])
            return 0

        lax.fori_loop(0, NCHUNK, chunk_body, 0)

    return k(center_idx, context_idx, neg_idx, center_flat, context_emb, ws)


def _tc_loss(pos_p, neg_p, sent_p):
    def body(pos_ref, neg_ref, sent_ref, out_ref):
        lane = lax.broadcasted_iota(jnp.int32, (128, 8), 0)
        grp = lax.broadcasted_iota(jnp.int32, (128, 8), 1)
        m = (lane // 16 == grp).astype(jnp.float32)
        pos = jnp.dot(pos_ref[...], m, preferred_element_type=jnp.float32)
        neg = jnp.dot(neg_ref[...], m, preferred_element_type=jnp.float32)
        sent = jnp.dot(sent_ref[...], m, preferred_element_type=jnp.float32)

        def log_sigmoid(x):
            return jnp.minimum(x, 0.0) - jnp.log1p(jnp.exp(-jnp.abs(x)))

        pos_val = log_sigmoid(pos)
        neg_val = log_sigmoid(-neg)
        sv = jax.nn.sigmoid(sent + INTERCEPT)
        sent_val = -jnp.abs(sv - 0.5)
        loss = pos_val + sent_val + neg_val
        out_ref[0, 0] = -jnp.sum(loss) / BATCH

    out = pl.pallas_call(
        body,
        out_shape=jax.ShapeDtypeStruct((1, 1), jnp.float32),
        out_specs=pl.BlockSpec(memory_space=pltpu.SMEM),
    )(pos_p, neg_p, sent_p)
    return out[0, 0]


def kernel(center_input, context_output, negative_samples, center_emb,
           context_emb, word_semantics):
    cidx = center_input.astype(jnp.int32)
    uidx = context_output.astype(jnp.int32)
    nidx = negative_samples.astype(jnp.int32).reshape(BATCH * NEG)
    cflat = center_emb.T.reshape(VOCAB * DIM)
    upad = jnp.pad(context_emb, ((0, 0), (0, PDIM - DIM)))
    pos_p, neg_p, sent_p = _sc_dots(cidx, uidx, nidx, cflat, upad,
                                    word_semantics)
    shp = (BATCH * 16 // 128, 128)
    return _tc_loss(pos_p.reshape(shp), neg_p.reshape(shp),
                    sent_p.reshape(shp))


# speedup vs baseline: 4.6286x; 4.6286x over previous
import functools

import jax
import jax.numpy as jnp
from jax import lax
from jax.experimental import pallas as pl
from jax.experimental.pallas import tpu as pltpu
from jax.experimental.pallas import tpu_sc as plsc

VOCAB = 1000000
DIM = 64
BATCH = 16384
NEG = 20
INTERCEPT = 1.1

NC = 2
NS = 16
NW = NC * NS
BPW = BATCH // NW
C = 32
NCHUNK = BPW // C
NEG_PER_CHUNK = C * NEG
NIDX_COLS = 128
NEG_GATHERS = NEG_PER_CHUNK // NIDX_COLS
NVREG = DIM // 16


def _sc_dots(center_idx, context_idx, neg_idx, center_emb, context_emb, ws):
    mesh = plsc.VectorSubcoreMesh(core_axis_name="c", subcore_axis_name="s")

    @functools.partial(
        pl.kernel,
        mesh=mesh,
        out_type=[jax.ShapeDtypeStruct((BATCH * 16,), jnp.float32)] * 3,
        compiler_params=pltpu.CompilerParams(use_tc_tiling_on_sc=False),
        scratch_types=[
            pltpu.VMEM((C,), jnp.int32),
            pltpu.VMEM((C,), jnp.int32),
            pltpu.VMEM((NEG_PER_CHUNK,), jnp.int32),
            pltpu.VMEM((C, DIM), jnp.float32),
            pltpu.VMEM((C, DIM), jnp.float32),
            pltpu.VMEM((NEG_PER_CHUNK, DIM), jnp.float32),
            pltpu.VMEM((C * 16,), jnp.float32),
            pltpu.VMEM((C * 16,), jnp.float32),
            pltpu.VMEM((C * 16,), jnp.float32),
            pltpu.VMEM((DIM,), jnp.float32),
            pltpu.SemaphoreType.DMA,
        ],
    )
    def k(cidx_hbm, uidx_hbm, nidx_hbm, cemb_hbm, uemb_hbm, ws_hbm,
          pos_out, neg_out, sent_out,
          cidx_v, uidx_v, nidx_v, vrows, urows, nrows,
          posb, negb, sentb, ws_v, sem):
        wid = lax.axis_index("s") * NC + lax.axis_index("c")
        pltpu.sync_copy(ws_hbm, ws_v)
        wsv = [ws_v[pl.ds(kk * 16, 16)] for kk in range(NVREG)]

        def chunk_body(c, _):
            base = wid * BPW + c * C
            pltpu.sync_copy(cidx_hbm.at[pl.ds(base, C)], cidx_v)
            pltpu.sync_copy(uidx_hbm.at[pl.ds(base, C)], uidx_v)
            pltpu.sync_copy(nidx_hbm.at[pl.ds(base * NEG, NEG_PER_CHUNK)],
                            nidx_v)
            cp1 = pltpu.async_copy(cemb_hbm.at[cidx_v], vrows, sem)
            cp2 = pltpu.async_copy(uemb_hbm.at[uidx_v], urows, sem)
            cps = [
                pltpu.async_copy(
                    uemb_hbm.at[nidx_v.at[pl.ds(j * NIDX_COLS, NIDX_COLS)]],
                    nrows.at[pl.ds(j * NIDX_COLS, NIDX_COLS)], sem)
                for j in range(NEG_GATHERS)
            ]
            cp1.wait()
            cp2.wait()
            for cp in cps:
                cp.wait()

            def b_body(b, _):
                v = [vrows[b, pl.ds(kk * 16, 16)] for kk in range(NVREG)]
                u = [urows[b, pl.ds(kk * 16, 16)] for kk in range(NVREG)]
                pos = v[0] * u[0]
                for kk in range(1, NVREG):
                    pos = pos + v[kk] * u[kk]
                sent = v[0] * wsv[0]
                for kk in range(1, NVREG):
                    sent = sent + v[kk] * wsv[kk]
                nacc = [nrows[b * NEG, pl.ds(kk * 16, 16)]
                        for kk in range(NVREG)]
                for n in range(1, NEG):
                    for kk in range(NVREG):
                        nacc[kk] = nacc[kk] + nrows[b * NEG + n,
                                                    pl.ds(kk * 16, 16)]
                neg = v[0] * nacc[0]
                for kk in range(1, NVREG):
                    neg = neg + v[kk] * nacc[kk]
                posb[pl.ds(b * 16, 16)] = pos
                sentb[pl.ds(b * 16, 16)] = sent
                negb[pl.ds(b * 16, 16)] = neg
                return 0

            lax.fori_loop(0, C, b_body, 0)
            pltpu.sync_copy(posb, pos_out.at[pl.ds(base * 16, C * 16)])
            pltpu.sync_copy(negb, neg_out.at[pl.ds(base * 16, C * 16)])
            pltpu.sync_copy(sentb, sent_out.at[pl.ds(base * 16, C * 16)])
            return 0

        lax.fori_loop(0, NCHUNK, chunk_body, 0)

    return k(center_idx, context_idx, neg_idx, center_emb, context_emb, ws)


def _tc_loss(pos_p, neg_p, sent_p):
    def body(pos_ref, neg_ref, sent_ref, out_ref):
        lane = lax.broadcasted_iota(jnp.int32, (128, 8), 0)
        grp = lax.broadcasted_iota(jnp.int32, (128, 8), 1)
        m = (lane // 16 == grp).astype(jnp.float32)
        pos = jnp.dot(pos_ref[...], m, preferred_element_type=jnp.float32)
        neg = jnp.dot(neg_ref[...], m, preferred_element_type=jnp.float32)
        sent = jnp.dot(sent_ref[...], m, preferred_element_type=jnp.float32)

        def log_sigmoid(x):
            return jnp.minimum(x, 0.0) - jnp.log1p(jnp.exp(-jnp.abs(x)))

        pos_val = log_sigmoid(pos)
        neg_val = log_sigmoid(-neg)
        sv = jax.nn.sigmoid(sent + INTERCEPT)
        sent_val = -jnp.abs(sv - 0.5)
        loss = pos_val + sent_val + neg_val
        out_ref[0, 0] = -jnp.sum(loss) / BATCH

    out = pl.pallas_call(
        body,
        out_shape=jax.ShapeDtypeStruct((1, 1), jnp.float32),
        out_specs=pl.BlockSpec(memory_space=pltpu.SMEM),
    )(pos_p, neg_p, sent_p)
    return out[0, 0]


def kernel(center_input, context_output, negative_samples, center_emb,
           context_emb, word_semantics):
    cidx = center_input.astype(jnp.int32)
    uidx = context_output.astype(jnp.int32)
    nidx = negative_samples.astype(jnp.int32).reshape(BATCH * NEG)
    pos_p, neg_p, sent_p = _sc_dots(cidx, uidx, nidx, center_emb,
                                    context_emb, word_semantics)
    shp = (BATCH * 16 // 128, 128)
    return _tc_loss(pos_p.reshape(shp), neg_p.reshape(shp),
                    sent_p.reshape(shp))


# speedup vs baseline: 4.9177x; 1.0625x over previous
import functools

import jax
import jax.numpy as jnp
from jax import lax
from jax.experimental import pallas as pl
from jax.experimental.pallas import tpu as pltpu
from jax.experimental.pallas import tpu_sc as plsc

VOCAB = 1000000
DIM = 64
BATCH = 16384
NEG = 20
INTERCEPT = 1.1

PDIM = 128
NC = 2
NS = 16
NW = NC * NS
BPW = BATCH // NW
C = 32
NCHUNK = BPW // C
NEG_PER_CHUNK = C * NEG
NIDX_COLS = 128
NEG_GATHERS = NEG_PER_CHUNK // NIDX_COLS
NVREG = DIM // 16

_MESH = dict(core_axis_name="c", subcore_axis_name="s")


def _sc_context(context_idx, neg_idx, context_emb):
    @functools.partial(
        pl.kernel,
        mesh=plsc.VectorSubcoreMesh(**_MESH),
        out_type=[jax.ShapeDtypeStruct((BATCH, PDIM), jnp.float32),
                  jax.ShapeDtypeStruct((BATCH * DIM,), jnp.float32)],
        scratch_types=[
            pltpu.VMEM((C,), jnp.int32),
            pltpu.VMEM((NEG_PER_CHUNK,), jnp.int32),
            pltpu.VMEM((C, PDIM), jnp.float32),
            pltpu.VMEM((NEG_PER_CHUNK, PDIM), jnp.float32),
            pltpu.VMEM((C * DIM,), jnp.float32),
            pltpu.SemaphoreType.DMA,
        ],
    )
    def k(uidx_hbm, nidx_hbm, uemb_hbm, u_out, s_out,
          uidx_v, nidx_v, urows, nrows, srows, sem):
        wid = lax.axis_index("s") * NC + lax.axis_index("c")

        def chunk_body(c, _):
            base = wid * BPW + c * C
            pltpu.sync_copy(uidx_hbm.at[pl.ds(base, C)], uidx_v)
            pltpu.sync_copy(nidx_hbm.at[pl.ds(base * NEG, NEG_PER_CHUNK)],
                            nidx_v)
            cp2 = pltpu.async_copy(uemb_hbm.at[uidx_v], urows, sem)
            cps = [
                pltpu.async_copy(
                    uemb_hbm.at[nidx_v.at[pl.ds(j * NIDX_COLS, NIDX_COLS)]],
                    nrows.at[pl.ds(j * NIDX_COLS, NIDX_COLS)], sem)
                for j in range(NEG_GATHERS)
            ]
            cp2.wait()
            for cp in cps:
                cp.wait()

            def b_body(b, _):
                nacc = [nrows[b * NEG, pl.ds(kk * 16, 16)]
                        for kk in range(NVREG)]
                for n in range(1, NEG):
                    for kk in range(NVREG):
                        nacc[kk] = nacc[kk] + nrows[b * NEG + n,
                                                    pl.ds(kk * 16, 16)]
                for kk in range(NVREG):
                    srows[pl.ds(b * DIM + kk * 16, 16)] = nacc[kk]
                return 0

            lax.fori_loop(0, C, b_body, 0)
            pltpu.sync_copy(urows, u_out.at[pl.ds(base, C)])
            pltpu.sync_copy(srows, s_out.at[pl.ds(base * DIM, C * DIM)])
            return 0

        lax.fori_loop(0, NCHUNK, chunk_body, 0)

    return k(context_idx, neg_idx, context_emb)


def _sc_dots(center_idx, center_emb, u_rows, s_rows, ws):
    @functools.partial(
        pl.kernel,
        mesh=plsc.VectorSubcoreMesh(**_MESH),
        out_type=[jax.ShapeDtypeStruct((BATCH * 16,), jnp.float32)] * 3,
        scratch_types=[
            pltpu.VMEM((C,), jnp.int32),
            pltpu.VMEM((C, PDIM), jnp.float32),
            pltpu.VMEM((C, PDIM), jnp.float32),
            pltpu.VMEM((C * DIM,), jnp.float32),
            pltpu.VMEM((C * 16,), jnp.float32),
            pltpu.VMEM((C * 16,), jnp.float32),
            pltpu.VMEM((C * 16,), jnp.float32),
            pltpu.VMEM((DIM,), jnp.float32),
            pltpu.SemaphoreType.DMA,
        ],
    )
    def k(cidx_hbm, cemb_hbm, u_hbm, s_hbm, ws_hbm,
          pos_out, neg_out, sent_out,
          cidx_v, vrows, ubuf, sbuf, posb, negb, sentb, ws_v, sem):
        wid = lax.axis_index("s") * NC + lax.axis_index("c")
        pltpu.sync_copy(ws_hbm, ws_v)
        wsv = [ws_v[pl.ds(kk * 16, 16)] for kk in range(NVREG)]

        def chunk_body(c, _):
            base = wid * BPW + c * C
            pltpu.sync_copy(cidx_hbm.at[pl.ds(base, C)], cidx_v)
            cp1 = pltpu.async_copy(cemb_hbm.at[cidx_v], vrows, sem)
            cp2 = pltpu.async_copy(u_hbm.at[pl.ds(base, C)], ubuf, sem)
            cp3 = pltpu.async_copy(s_hbm.at[pl.ds(base * DIM, C * DIM)],
                                   sbuf, sem)
            cp1.wait()
            cp2.wait()
            cp3.wait()

            def b_body(b, _):
                v = [vrows[b, pl.ds(kk * 16, 16)] for kk in range(NVREG)]
                u = [ubuf[b, pl.ds(kk * 16, 16)] for kk in range(NVREG)]
                s = [sbuf[pl.ds(b * DIM + kk * 16, 16)]
                     for kk in range(NVREG)]
                pos = v[0] * u[0]
                sent = v[0] * wsv[0]
                neg = v[0] * s[0]
                for kk in range(1, NVREG):
                    pos = pos + v[kk] * u[kk]
                    sent = sent + v[kk] * wsv[kk]
                    neg = neg + v[kk] * s[kk]
                posb[pl.ds(b * 16, 16)] = pos
                sentb[pl.ds(b * 16, 16)] = sent
                negb[pl.ds(b * 16, 16)] = neg
                return 0

            lax.fori_loop(0, C, b_body, 0)
            pltpu.sync_copy(posb, pos_out.at[pl.ds(base * 16, C * 16)])
            pltpu.sync_copy(negb, neg_out.at[pl.ds(base * 16, C * 16)])
            pltpu.sync_copy(sentb, sent_out.at[pl.ds(base * 16, C * 16)])
            return 0

        lax.fori_loop(0, NCHUNK, chunk_body, 0)

    return k(center_idx, center_emb, u_rows, s_rows, ws)


def _tc_loss(pos_p, neg_p, sent_p):
    def body(pos_ref, neg_ref, sent_ref, out_ref):
        lane = lax.broadcasted_iota(jnp.int32, (128, 8), 0)
        grp = lax.broadcasted_iota(jnp.int32, (128, 8), 1)
        m = (lane // 16 == grp).astype(jnp.float32)
        pos = jnp.dot(pos_ref[...], m, preferred_element_type=jnp.float32)
        neg = jnp.dot(neg_ref[...], m, preferred_element_type=jnp.float32)
        sent = jnp.dot(sent_ref[...], m, preferred_element_type=jnp.float32)

        def log_sigmoid(x):
            return jnp.minimum(x, 0.0) - jnp.log1p(jnp.exp(-jnp.abs(x)))

        pos_val = log_sigmoid(pos)
        neg_val = log_sigmoid(-neg)
        sv = jax.nn.sigmoid(sent + INTERCEPT)
        sent_val = -jnp.abs(sv - 0.5)
        loss = pos_val + sent_val + neg_val
        out_ref[0, 0] = -jnp.sum(loss) / BATCH

    out = pl.pallas_call(
        body,
        out_shape=jax.ShapeDtypeStruct((1, 1), jnp.float32),
        out_specs=pl.BlockSpec(memory_space=pltpu.SMEM),
    )(pos_p, neg_p, sent_p)
    return out[0, 0]


def kernel(center_input, context_output, negative_samples, center_emb,
           context_emb, word_semantics):
    cidx = center_input.astype(jnp.int32)
    uidx = context_output.astype(jnp.int32)
    nidx = negative_samples.astype(jnp.int32).reshape(BATCH * NEG)
    cpad = jnp.pad(center_emb, ((0, 0), (0, PDIM - DIM)))
    upad = jnp.pad(context_emb, ((0, 0), (0, PDIM - DIM)))
    u_rows, s_rows = _sc_context(uidx, nidx, upad)
    pos_p, neg_p, sent_p = _sc_dots(cidx, cpad, u_rows, s_rows,
                                    word_semantics)
    shp = (BATCH * 16 // 128, 128)
    return _tc_loss(pos_p.reshape(shp), neg_p.reshape(shp),
                    sent_p.reshape(shp))


# speedup vs baseline: 8.5021x; 1.7289x over previous
import functools

import jax
import jax.numpy as jnp
from jax import lax
from jax.experimental import pallas as pl
from jax.experimental.pallas import tpu as pltpu
from jax.experimental.pallas import tpu_sc as plsc

VOCAB = 1000000
DIM = 64
BATCH = 16384
NEG = 20
INTERCEPT = 1.1

PDIM = 128
NC = 2
NS = 16
NW = NC * NS
BPW = BATCH // NW
C = 32
NCHUNK = BPW // C
NEG_PER_CHUNK = C * NEG
NIDX_COLS = 128
NEG_GATHERS = NEG_PER_CHUNK // NIDX_COLS
NVREG = DIM // 16

_MESH = dict(core_axis_name="c", subcore_axis_name="s")


TBLK = 8192


def _tc_pad(emb_t):
    def body(x_ref, o_ref):
        xt = jnp.transpose(x_ref[...], (1, 0))
        z = jnp.zeros((TBLK, PDIM - DIM), jnp.float32)
        o_ref[...] = jnp.concatenate([xt, z], axis=1)

    return pl.pallas_call(
        body,
        grid=((VOCAB + TBLK - 1) // TBLK,),
        in_specs=[pl.BlockSpec((DIM, TBLK), lambda i: (0, i))],
        out_specs=pl.BlockSpec((TBLK, PDIM), lambda i: (i, 0)),
        out_shape=jax.ShapeDtypeStruct((VOCAB, PDIM), jnp.float32),
    )(emb_t)


def _sc_context(context_idx, neg_idx, context_emb):
    @functools.partial(
        pl.kernel,
        mesh=plsc.VectorSubcoreMesh(**_MESH),
        out_type=[jax.ShapeDtypeStruct((BATCH, PDIM), jnp.float32),
                  jax.ShapeDtypeStruct((BATCH * DIM,), jnp.float32)],
        scratch_types=[
            pltpu.VMEM((C,), jnp.int32),
            pltpu.VMEM((NEG_PER_CHUNK,), jnp.int32),
            pltpu.VMEM((C, PDIM), jnp.float32),
            pltpu.VMEM((NEG_PER_CHUNK, PDIM), jnp.float32),
            pltpu.VMEM((C * DIM,), jnp.float32),
            pltpu.SemaphoreType.DMA,
        ],
    )
    def k(uidx_hbm, nidx_hbm, uemb_hbm, u_out, s_out,
          uidx_v, nidx_v, urows, nrows, srows, sem):
        wid = lax.axis_index("s") * NC + lax.axis_index("c")

        def chunk_body(c, _):
            base = wid * BPW + c * C
            pltpu.sync_copy(uidx_hbm.at[pl.ds(base, C)], uidx_v)
            pltpu.sync_copy(nidx_hbm.at[pl.ds(base * NEG, NEG_PER_CHUNK)],
                            nidx_v)
            cp2 = pltpu.async_copy(uemb_hbm.at[uidx_v], urows, sem)
            cps = [
                pltpu.async_copy(
                    uemb_hbm.at[nidx_v.at[pl.ds(j * NIDX_COLS, NIDX_COLS)]],
                    nrows.at[pl.ds(j * NIDX_COLS, NIDX_COLS)], sem)
                for j in range(NEG_GATHERS)
            ]
            cp2.wait()
            for cp in cps:
                cp.wait()

            def b_body(b, _):
                nacc = [nrows[b * NEG, pl.ds(kk * 16, 16)]
                        for kk in range(NVREG)]
                for n in range(1, NEG):
                    for kk in range(NVREG):
                        nacc[kk] = nacc[kk] + nrows[b * NEG + n,
                                                    pl.ds(kk * 16, 16)]
                for kk in range(NVREG):
                    srows[pl.ds(b * DIM + kk * 16, 16)] = nacc[kk]
                return 0

            lax.fori_loop(0, C, b_body, 0)
            pltpu.sync_copy(urows, u_out.at[pl.ds(base, C)])
            pltpu.sync_copy(srows, s_out.at[pl.ds(base * DIM, C * DIM)])
            return 0

        lax.fori_loop(0, NCHUNK, chunk_body, 0)

    return k(context_idx, neg_idx, context_emb)


def _sc_dots(center_idx, center_emb, u_rows, s_rows, ws):
    @functools.partial(
        pl.kernel,
        mesh=plsc.VectorSubcoreMesh(**_MESH),
        out_type=[jax.ShapeDtypeStruct((BATCH * 16,), jnp.float32)] * 3,
        scratch_types=[
            pltpu.VMEM((C,), jnp.int32),
            pltpu.VMEM((C, PDIM), jnp.float32),
            pltpu.VMEM((C, PDIM), jnp.float32),
            pltpu.VMEM((C * DIM,), jnp.float32),
            pltpu.VMEM((C * 16,), jnp.float32),
            pltpu.VMEM((C * 16,), jnp.float32),
            pltpu.VMEM((C * 16,), jnp.float32),
            pltpu.VMEM((DIM,), jnp.float32),
            pltpu.SemaphoreType.DMA,
        ],
    )
    def k(cidx_hbm, cemb_hbm, u_hbm, s_hbm, ws_hbm,
          pos_out, neg_out, sent_out,
          cidx_v, vrows, ubuf, sbuf, posb, negb, sentb, ws_v, sem):
        wid = lax.axis_index("s") * NC + lax.axis_index("c")
        pltpu.sync_copy(ws_hbm, ws_v)
        wsv = [ws_v[pl.ds(kk * 16, 16)] for kk in range(NVREG)]

        def chunk_body(c, _):
            base = wid * BPW + c * C
            pltpu.sync_copy(cidx_hbm.at[pl.ds(base, C)], cidx_v)
            cp1 = pltpu.async_copy(cemb_hbm.at[cidx_v], vrows, sem)
            cp2 = pltpu.async_copy(u_hbm.at[pl.ds(base, C)], ubuf, sem)
            cp3 = pltpu.async_copy(s_hbm.at[pl.ds(base * DIM, C * DIM)],
                                   sbuf, sem)
            cp1.wait()
            cp2.wait()
            cp3.wait()

            def b_body(b, _):
                v = [vrows[b, pl.ds(kk * 16, 16)] for kk in range(NVREG)]
                u = [ubuf[b, pl.ds(kk * 16, 16)] for kk in range(NVREG)]
                s = [sbuf[pl.ds(b * DIM + kk * 16, 16)]
                     for kk in range(NVREG)]
                pos = v[0] * u[0]
                sent = v[0] * wsv[0]
                neg = v[0] * s[0]
                for kk in range(1, NVREG):
                    pos = pos + v[kk] * u[kk]
                    sent = sent + v[kk] * wsv[kk]
                    neg = neg + v[kk] * s[kk]
                posb[pl.ds(b * 16, 16)] = pos
                sentb[pl.ds(b * 16, 16)] = sent
                negb[pl.ds(b * 16, 16)] = neg
                return 0

            lax.fori_loop(0, C, b_body, 0)
            pltpu.sync_copy(posb, pos_out.at[pl.ds(base * 16, C * 16)])
            pltpu.sync_copy(negb, neg_out.at[pl.ds(base * 16, C * 16)])
            pltpu.sync_copy(sentb, sent_out.at[pl.ds(base * 16, C * 16)])
            return 0

        lax.fori_loop(0, NCHUNK, chunk_body, 0)

    return k(center_idx, center_emb, u_rows, s_rows, ws)


def _tc_loss(pos_p, neg_p, sent_p):
    def body(pos_ref, neg_ref, sent_ref, out_ref):
        lane = lax.broadcasted_iota(jnp.int32, (128, 8), 0)
        grp = lax.broadcasted_iota(jnp.int32, (128, 8), 1)
        m = (lane // 16 == grp).astype(jnp.float32)
        pos = jnp.dot(pos_ref[...], m, preferred_element_type=jnp.float32)
        neg = jnp.dot(neg_ref[...], m, preferred_element_type=jnp.float32)
        sent = jnp.dot(sent_ref[...], m, preferred_element_type=jnp.float32)

        def log_sigmoid(x):
            return jnp.minimum(x, 0.0) - jnp.log1p(jnp.exp(-jnp.abs(x)))

        pos_val = log_sigmoid(pos)
        neg_val = log_sigmoid(-neg)
        sv = jax.nn.sigmoid(sent + INTERCEPT)
        sent_val = -jnp.abs(sv - 0.5)
        loss = pos_val + sent_val + neg_val
        out_ref[0, 0] = -jnp.sum(loss) / BATCH

    out = pl.pallas_call(
        body,
        out_shape=jax.ShapeDtypeStruct((1, 1), jnp.float32),
        out_specs=pl.BlockSpec(memory_space=pltpu.SMEM),
    )(pos_p, neg_p, sent_p)
    return out[0, 0]


def kernel(center_input, context_output, negative_samples, center_emb,
           context_emb, word_semantics):
    cidx = center_input.astype(jnp.int32)
    uidx = context_output.astype(jnp.int32)
    nidx = negative_samples.astype(jnp.int32).reshape(BATCH * NEG)
    upad = _tc_pad(context_emb.T)
    cpad = _tc_pad(center_emb.T)
    u_rows, s_rows = _sc_context(uidx, nidx, upad)
    pos_p, neg_p, sent_p = _sc_dots(cidx, cpad, u_rows, s_rows,
                                    word_semantics)
    shp = (BATCH * 16 // 128, 128)
    return _tc_loss(pos_p.reshape(shp), neg_p.reshape(shp),
                    sent_p.reshape(shp))


# speedup vs baseline: 8.9558x; 1.0534x over previous
import functools

import jax
import jax.numpy as jnp
from jax import lax
from jax.experimental import pallas as pl
from jax.experimental.pallas import tpu as pltpu
from jax.experimental.pallas import tpu_sc as plsc

VOCAB = 1000000
DIM = 64
BATCH = 16384
NEG = 20
INTERCEPT = 1.1

PDIM = 128
NC = 2
NS = 16
NW = NC * NS
BPW = BATCH // NW
C = 32
NCHUNK = BPW // C
NEG_PER_CHUNK = C * NEG
NIDX_COLS = 128
NEG_GATHERS = NEG_PER_CHUNK // NIDX_COLS
NVREG = DIM // 16

_MESH = dict(core_axis_name="c", subcore_axis_name="s")


TBLK = 16384


def _tc_pad(emb_t):
    def body(x_ref, o_ref):
        xt = jnp.transpose(x_ref[...], (1, 0))
        z = jnp.zeros((TBLK, PDIM - DIM), jnp.float32)
        o_ref[...] = jnp.concatenate([xt, z], axis=1)

    return pl.pallas_call(
        body,
        grid=((VOCAB + TBLK - 1) // TBLK,),
        in_specs=[pl.BlockSpec((DIM, TBLK), lambda i: (0, i))],
        out_specs=pl.BlockSpec((TBLK, PDIM), lambda i: (i, 0)),
        out_shape=jax.ShapeDtypeStruct((VOCAB, PDIM), jnp.float32),
    )(emb_t)


def _sc_context(context_idx, neg_idx, context_emb):
    @functools.partial(
        pl.kernel,
        mesh=plsc.VectorSubcoreMesh(**_MESH),
        out_type=[jax.ShapeDtypeStruct((BATCH, PDIM), jnp.float32),
                  jax.ShapeDtypeStruct((BATCH * DIM,), jnp.float32)],
        scratch_types=[
            pltpu.VMEM((C,), jnp.int32),
            pltpu.VMEM((NEG_PER_CHUNK,), jnp.int32),
            pltpu.VMEM((C, PDIM), jnp.float32),
            pltpu.VMEM((NEG_PER_CHUNK, PDIM), jnp.float32),
            pltpu.VMEM((C * DIM,), jnp.float32),
            pltpu.SemaphoreType.DMA,
        ],
    )
    def k(uidx_hbm, nidx_hbm, uemb_hbm, u_out, s_out,
          uidx_v, nidx_v, urows, nrows, srows, sem):
        wid = lax.axis_index("s") * NC + lax.axis_index("c")

        def chunk_body(c, _):
            base = wid * BPW + c * C
            pltpu.sync_copy(uidx_hbm.at[pl.ds(base, C)], uidx_v)
            pltpu.sync_copy(nidx_hbm.at[pl.ds(base * NEG, NEG_PER_CHUNK)],
                            nidx_v)
            cp2 = pltpu.async_copy(uemb_hbm.at[uidx_v], urows, sem)
            cps = [
                pltpu.async_copy(
                    uemb_hbm.at[nidx_v.at[pl.ds(j * NIDX_COLS, NIDX_COLS)]],
                    nrows.at[pl.ds(j * NIDX_COLS, NIDX_COLS)], sem)
                for j in range(NEG_GATHERS)
            ]
            cp2.wait()
            for cp in cps:
                cp.wait()

            def b_body(b, _):
                nacc = [nrows[b * NEG, pl.ds(kk * 16, 16)]
                        for kk in range(NVREG)]
                for n in range(1, NEG):
                    for kk in range(NVREG):
                        nacc[kk] = nacc[kk] + nrows[b * NEG + n,
                                                    pl.ds(kk * 16, 16)]
                for kk in range(NVREG):
                    srows[pl.ds(b * DIM + kk * 16, 16)] = nacc[kk]
                return 0

            lax.fori_loop(0, C, b_body, 0)
            pltpu.sync_copy(urows, u_out.at[pl.ds(base, C)])
            pltpu.sync_copy(srows, s_out.at[pl.ds(base * DIM, C * DIM)])
            return 0

        lax.fori_loop(0, NCHUNK, chunk_body, 0)

    return k(context_idx, neg_idx, context_emb)


def _sc_dots(center_idx, center_emb, u_rows, s_rows, ws):
    @functools.partial(
        pl.kernel,
        mesh=plsc.VectorSubcoreMesh(**_MESH),
        out_type=[jax.ShapeDtypeStruct((BATCH * 16,), jnp.float32)] * 3,
        scratch_types=[
            pltpu.VMEM((C,), jnp.int32),
            pltpu.VMEM((C, PDIM), jnp.float32),
            pltpu.VMEM((C, PDIM), jnp.float32),
            pltpu.VMEM((C * DIM,), jnp.float32),
            pltpu.VMEM((C * 16,), jnp.float32),
            pltpu.VMEM((C * 16,), jnp.float32),
            pltpu.VMEM((C * 16,), jnp.float32),
            pltpu.VMEM((DIM,), jnp.float32),
            pltpu.SemaphoreType.DMA,
        ],
    )
    def k(cidx_hbm, cemb_hbm, u_hbm, s_hbm, ws_hbm,
          pos_out, neg_out, sent_out,
          cidx_v, vrows, ubuf, sbuf, posb, negb, sentb, ws_v, sem):
        wid = lax.axis_index("s") * NC + lax.axis_index("c")
        pltpu.sync_copy(ws_hbm, ws_v)
        wsv = [ws_v[pl.ds(kk * 16, 16)] for kk in range(NVREG)]

        def chunk_body(c, _):
            base = wid * BPW + c * C
            pltpu.sync_copy(cidx_hbm.at[pl.ds(base, C)], cidx_v)
            cp1 = pltpu.async_copy(cemb_hbm.at[cidx_v], vrows, sem)
            cp2 = pltpu.async_copy(u_hbm.at[pl.ds(base, C)], ubuf, sem)
            cp3 = pltpu.async_copy(s_hbm.at[pl.ds(base * DIM, C * DIM)],
                                   sbuf, sem)
            cp1.wait()
            cp2.wait()
            cp3.wait()

            def b_body(b, _):
                v = [vrows[b, pl.ds(kk * 16, 16)] for kk in range(NVREG)]
                u = [ubuf[b, pl.ds(kk * 16, 16)] for kk in range(NVREG)]
                s = [sbuf[pl.ds(b * DIM + kk * 16, 16)]
                     for kk in range(NVREG)]
                pos = v[0] * u[0]
                sent = v[0] * wsv[0]
                neg = v[0] * s[0]
                for kk in range(1, NVREG):
                    pos = pos + v[kk] * u[kk]
                    sent = sent + v[kk] * wsv[kk]
                    neg = neg + v[kk] * s[kk]
                posb[pl.ds(b * 16, 16)] = pos
                sentb[pl.ds(b * 16, 16)] = sent
                negb[pl.ds(b * 16, 16)] = neg
                return 0

            lax.fori_loop(0, C, b_body, 0)
            pltpu.sync_copy(posb, pos_out.at[pl.ds(base * 16, C * 16)])
            pltpu.sync_copy(negb, neg_out.at[pl.ds(base * 16, C * 16)])
            pltpu.sync_copy(sentb, sent_out.at[pl.ds(base * 16, C * 16)])
            return 0

        lax.fori_loop(0, NCHUNK, chunk_body, 0)

    return k(center_idx, center_emb, u_rows, s_rows, ws)


def _tc_loss(pos_p, neg_p, sent_p):
    def body(pos_ref, neg_ref, sent_ref, out_ref):
        lane = lax.broadcasted_iota(jnp.int32, (128, 8), 0)
        grp = lax.broadcasted_iota(jnp.int32, (128, 8), 1)
        m = (lane // 16 == grp).astype(jnp.float32)
        pos = jnp.dot(pos_ref[...], m, preferred_element_type=jnp.float32)
        neg = jnp.dot(neg_ref[...], m, preferred_element_type=jnp.float32)
        sent = jnp.dot(sent_ref[...], m, preferred_element_type=jnp.float32)

        def log_sigmoid(x):
            return jnp.minimum(x, 0.0) - jnp.log1p(jnp.exp(-jnp.abs(x)))

        pos_val = log_sigmoid(pos)
        neg_val = log_sigmoid(-neg)
        sv = jax.nn.sigmoid(sent + INTERCEPT)
        sent_val = -jnp.abs(sv - 0.5)
        loss = pos_val + sent_val + neg_val
        out_ref[0, 0] = -jnp.sum(loss) / BATCH

    out = pl.pallas_call(
        body,
        out_shape=jax.ShapeDtypeStruct((1, 1), jnp.float32),
        out_specs=pl.BlockSpec(memory_space=pltpu.SMEM),
    )(pos_p, neg_p, sent_p)
    return out[0, 0]


def kernel(center_input, context_output, negative_samples, center_emb,
           context_emb, word_semantics):
    cidx = center_input.astype(jnp.int32)
    uidx = context_output.astype(jnp.int32)
    nidx = negative_samples.astype(jnp.int32).reshape(BATCH * NEG)
    upad = _tc_pad(context_emb.T)
    cpad = _tc_pad(center_emb.T)
    u_rows, s_rows = _sc_context(uidx, nidx, upad)
    pos_p, neg_p, sent_p = _sc_dots(cidx, cpad, u_rows, s_rows,
                                    word_semantics)
    shp = (BATCH * 16 // 128, 128)
    return _tc_loss(pos_p.reshape(shp), neg_p.reshape(shp),
                    sent_p.reshape(shp))


# speedup vs baseline: 9.1279x; 1.0192x over previous
import functools

import jax
import jax.numpy as jnp
from jax import lax
from jax.experimental import pallas as pl
from jax.experimental.pallas import tpu as pltpu
from jax.experimental.pallas import tpu_sc as plsc

VOCAB = 1000000
DIM = 64
BATCH = 16384
NEG = 20
INTERCEPT = 1.1

PDIM = 128
NC = 2
NS = 16
NW = NC * NS
BPW = BATCH // NW
C = 32
NCHUNK = BPW // C
NEG_PER_CHUNK = C * NEG
NIDX_COLS = 128
NEG_GATHERS = NEG_PER_CHUNK // NIDX_COLS
NVREG = DIM // 16

_MESH = dict(core_axis_name="c", subcore_axis_name="s")


TBLK = 32768


def _tc_pad(emb_t):
    def body(x_ref, o_ref):
        xt = jnp.transpose(x_ref[...], (1, 0))
        z = jnp.zeros((TBLK, PDIM - DIM), jnp.float32)
        o_ref[...] = jnp.concatenate([xt, z], axis=1)

    return pl.pallas_call(
        body,
        grid=((VOCAB + TBLK - 1) // TBLK,),
        in_specs=[pl.BlockSpec((DIM, TBLK), lambda i: (0, i))],
        out_specs=pl.BlockSpec((TBLK, PDIM), lambda i: (i, 0)),
        out_shape=jax.ShapeDtypeStruct((VOCAB, PDIM), jnp.float32),
    )(emb_t)


def _sc_context(context_idx, neg_idx, context_emb):
    @functools.partial(
        pl.kernel,
        mesh=plsc.VectorSubcoreMesh(**_MESH),
        out_type=[jax.ShapeDtypeStruct((BATCH, PDIM), jnp.float32),
                  jax.ShapeDtypeStruct((BATCH * DIM,), jnp.float32)],
        scratch_types=[
            pltpu.VMEM((C,), jnp.int32),
            pltpu.VMEM((NEG_PER_CHUNK,), jnp.int32),
            pltpu.VMEM((C, PDIM), jnp.float32),
            pltpu.VMEM((NEG_PER_CHUNK, PDIM), jnp.float32),
            pltpu.VMEM((C * DIM,), jnp.float32),
            pltpu.SemaphoreType.DMA,
        ],
    )
    def k(uidx_hbm, nidx_hbm, uemb_hbm, u_out, s_out,
          uidx_v, nidx_v, urows, nrows, srows, sem):
        wid = lax.axis_index("s") * NC + lax.axis_index("c")

        def chunk_body(c, _):
            base = wid * BPW + c * C
            pltpu.sync_copy(uidx_hbm.at[pl.ds(base, C)], uidx_v)
            pltpu.sync_copy(nidx_hbm.at[pl.ds(base * NEG, NEG_PER_CHUNK)],
                            nidx_v)
            cp2 = pltpu.async_copy(uemb_hbm.at[uidx_v], urows, sem)
            cps = [
                pltpu.async_copy(
                    uemb_hbm.at[nidx_v.at[pl.ds(j * NIDX_COLS, NIDX_COLS)]],
                    nrows.at[pl.ds(j * NIDX_COLS, NIDX_COLS)], sem)
                for j in range(NEG_GATHERS)
            ]
            cp2.wait()
            for cp in cps:
                cp.wait()

            def b_body(b, _):
                nacc = [nrows[b * NEG, pl.ds(kk * 16, 16)]
                        for kk in range(NVREG)]
                for n in range(1, NEG):
                    for kk in range(NVREG):
                        nacc[kk] = nacc[kk] + nrows[b * NEG + n,
                                                    pl.ds(kk * 16, 16)]
                for kk in range(NVREG):
                    srows[pl.ds(b * DIM + kk * 16, 16)] = nacc[kk]
                return 0

            lax.fori_loop(0, C, b_body, 0)
            pltpu.sync_copy(urows, u_out.at[pl.ds(base, C)])
            pltpu.sync_copy(srows, s_out.at[pl.ds(base * DIM, C * DIM)])
            return 0

        lax.fori_loop(0, NCHUNK, chunk_body, 0)

    return k(context_idx, neg_idx, context_emb)


def _sc_dots(center_idx, center_emb, u_rows, s_rows, ws):
    @functools.partial(
        pl.kernel,
        mesh=plsc.VectorSubcoreMesh(**_MESH),
        out_type=[jax.ShapeDtypeStruct((BATCH * 16,), jnp.float32)] * 3,
        scratch_types=[
            pltpu.VMEM((C,), jnp.int32),
            pltpu.VMEM((C, PDIM), jnp.float32),
            pltpu.VMEM((C, PDIM), jnp.float32),
            pltpu.VMEM((C * DIM,), jnp.float32),
            pltpu.VMEM((C * 16,), jnp.float32),
            pltpu.VMEM((C * 16,), jnp.float32),
            pltpu.VMEM((C * 16,), jnp.float32),
            pltpu.VMEM((DIM,), jnp.float32),
            pltpu.SemaphoreType.DMA,
        ],
    )
    def k(cidx_hbm, cemb_hbm, u_hbm, s_hbm, ws_hbm,
          pos_out, neg_out, sent_out,
          cidx_v, vrows, ubuf, sbuf, posb, negb, sentb, ws_v, sem):
        wid = lax.axis_index("s") * NC + lax.axis_index("c")
        pltpu.sync_copy(ws_hbm, ws_v)
        wsv = [ws_v[pl.ds(kk * 16, 16)] for kk in range(NVREG)]

        def chunk_body(c, _):
            base = wid * BPW + c * C
            pltpu.sync_copy(cidx_hbm.at[pl.ds(base, C)], cidx_v)
            cp1 = pltpu.async_copy(cemb_hbm.at[cidx_v], vrows, sem)
            cp2 = pltpu.async_copy(u_hbm.at[pl.ds(base, C)], ubuf, sem)
            cp3 = pltpu.async_copy(s_hbm.at[pl.ds(base * DIM, C * DIM)],
                                   sbuf, sem)
            cp1.wait()
            cp2.wait()
            cp3.wait()

            def b_body(b, _):
                v = [vrows[b, pl.ds(kk * 16, 16)] for kk in range(NVREG)]
                u = [ubuf[b, pl.ds(kk * 16, 16)] for kk in range(NVREG)]
                s = [sbuf[pl.ds(b * DIM + kk * 16, 16)]
                     for kk in range(NVREG)]
                pos = v[0] * u[0]
                sent = v[0] * wsv[0]
                neg = v[0] * s[0]
                for kk in range(1, NVREG):
                    pos = pos + v[kk] * u[kk]
                    sent = sent + v[kk] * wsv[kk]
                    neg = neg + v[kk] * s[kk]
                posb[pl.ds(b * 16, 16)] = pos
                sentb[pl.ds(b * 16, 16)] = sent
                negb[pl.ds(b * 16, 16)] = neg
                return 0

            lax.fori_loop(0, C, b_body, 0)
            pltpu.sync_copy(posb, pos_out.at[pl.ds(base * 16, C * 16)])
            pltpu.sync_copy(negb, neg_out.at[pl.ds(base * 16, C * 16)])
            pltpu.sync_copy(sentb, sent_out.at[pl.ds(base * 16, C * 16)])
            return 0

        lax.fori_loop(0, NCHUNK, chunk_body, 0)

    return k(center_idx, center_emb, u_rows, s_rows, ws)


def _tc_loss(pos_p, neg_p, sent_p):
    def body(pos_ref, neg_ref, sent_ref, out_ref):
        lane = lax.broadcasted_iota(jnp.int32, (128, 8), 0)
        grp = lax.broadcasted_iota(jnp.int32, (128, 8), 1)
        m = (lane // 16 == grp).astype(jnp.float32)
        pos = jnp.dot(pos_ref[...], m, preferred_element_type=jnp.float32)
        neg = jnp.dot(neg_ref[...], m, preferred_element_type=jnp.float32)
        sent = jnp.dot(sent_ref[...], m, preferred_element_type=jnp.float32)

        def log_sigmoid(x):
            return jnp.minimum(x, 0.0) - jnp.log1p(jnp.exp(-jnp.abs(x)))

        pos_val = log_sigmoid(pos)
        neg_val = log_sigmoid(-neg)
        sv = jax.nn.sigmoid(sent + INTERCEPT)
        sent_val = -jnp.abs(sv - 0.5)
        loss = pos_val + sent_val + neg_val
        out_ref[0, 0] = -jnp.sum(loss) / BATCH

    out = pl.pallas_call(
        body,
        out_shape=jax.ShapeDtypeStruct((1, 1), jnp.float32),
        out_specs=pl.BlockSpec(memory_space=pltpu.SMEM),
    )(pos_p, neg_p, sent_p)
    return out[0, 0]


def kernel(center_input, context_output, negative_samples, center_emb,
           context_emb, word_semantics):
    cidx = center_input.astype(jnp.int32)
    uidx = context_output.astype(jnp.int32)
    nidx = negative_samples.astype(jnp.int32).reshape(BATCH * NEG)
    upad = _tc_pad(context_emb.T)
    cpad = _tc_pad(center_emb.T)
    u_rows, s_rows = _sc_context(uidx, nidx, upad)
    pos_p, neg_p, sent_p = _sc_dots(cidx, cpad, u_rows, s_rows,
                                    word_semantics)
    shp = (BATCH * 16 // 128, 128)
    return _tc_loss(pos_p.reshape(shp), neg_p.reshape(shp),
                    sent_p.reshape(shp))


# speedup vs baseline: 9.1417x; 1.0015x over previous
import functools

import jax
import jax.numpy as jnp
from jax import lax
from jax.experimental import pallas as pl
from jax.experimental.pallas import tpu as pltpu
from jax.experimental.pallas import tpu_sc as plsc

VOCAB = 1000000
DIM = 64
BATCH = 16384
NEG = 20
INTERCEPT = 1.1

PDIM = 128
NC = 2
NS = 16
NW = NC * NS
BPW = BATCH // NW
C = 32
NCHUNK = BPW // C
NEG_PER_CHUNK = C * NEG
NIDX_COLS = 128
NEG_GATHERS = NEG_PER_CHUNK // NIDX_COLS
NVREG = DIM // 16

_MESH = dict(core_axis_name="c", subcore_axis_name="s")


TBLK = 36864


def _tc_pad(emb_t):
    def body(x_ref, o_ref):
        xt = jnp.transpose(x_ref[...], (1, 0))
        z = jnp.zeros((TBLK, PDIM - DIM), jnp.float32)
        o_ref[...] = jnp.concatenate([xt, z], axis=1)

    return pl.pallas_call(
        body,
        grid=((VOCAB + TBLK - 1) // TBLK,),
        in_specs=[pl.BlockSpec((DIM, TBLK), lambda i: (0, i))],
        out_specs=pl.BlockSpec((TBLK, PDIM), lambda i: (i, 0)),
        out_shape=jax.ShapeDtypeStruct((VOCAB, PDIM), jnp.float32),
    )(emb_t)


def _sc_context(context_idx, neg_idx, context_emb):
    @functools.partial(
        pl.kernel,
        mesh=plsc.VectorSubcoreMesh(**_MESH),
        out_type=[jax.ShapeDtypeStruct((BATCH, PDIM), jnp.float32),
                  jax.ShapeDtypeStruct((BATCH * DIM,), jnp.float32)],
        scratch_types=[
            pltpu.VMEM((C,), jnp.int32),
            pltpu.VMEM((NEG_PER_CHUNK,), jnp.int32),
            pltpu.VMEM((C, PDIM), jnp.float32),
            pltpu.VMEM((NEG_PER_CHUNK, PDIM), jnp.float32),
            pltpu.VMEM((C * DIM,), jnp.float32),
            pltpu.SemaphoreType.DMA,
        ],
    )
    def k(uidx_hbm, nidx_hbm, uemb_hbm, u_out, s_out,
          uidx_v, nidx_v, urows, nrows, srows, sem):
        wid = lax.axis_index("s") * NC + lax.axis_index("c")

        def chunk_body(c, _):
            base = wid * BPW + c * C
            pltpu.sync_copy(uidx_hbm.at[pl.ds(base, C)], uidx_v)
            pltpu.sync_copy(nidx_hbm.at[pl.ds(base * NEG, NEG_PER_CHUNK)],
                            nidx_v)
            cp2 = pltpu.async_copy(uemb_hbm.at[uidx_v], urows, sem)
            cps = [
                pltpu.async_copy(
                    uemb_hbm.at[nidx_v.at[pl.ds(j * NIDX_COLS, NIDX_COLS)]],
                    nrows.at[pl.ds(j * NIDX_COLS, NIDX_COLS)], sem)
                for j in range(NEG_GATHERS)
            ]
            cp2.wait()
            for cp in cps:
                cp.wait()

            def b_body(b, _):
                nacc = [nrows[b * NEG, pl.ds(kk * 16, 16)]
                        for kk in range(NVREG)]
                for n in range(1, NEG):
                    for kk in range(NVREG):
                        nacc[kk] = nacc[kk] + nrows[b * NEG + n,
                                                    pl.ds(kk * 16, 16)]
                for kk in range(NVREG):
                    srows[pl.ds(b * DIM + kk * 16, 16)] = nacc[kk]
                return 0

            lax.fori_loop(0, C, b_body, 0)
            pltpu.sync_copy(urows, u_out.at[pl.ds(base, C)])
            pltpu.sync_copy(srows, s_out.at[pl.ds(base * DIM, C * DIM)])
            return 0

        lax.fori_loop(0, NCHUNK, chunk_body, 0)

    return k(context_idx, neg_idx, context_emb)


def _sc_dots(center_idx, center_emb, u_rows, s_rows, ws):
    @functools.partial(
        pl.kernel,
        mesh=plsc.VectorSubcoreMesh(**_MESH),
        out_type=[jax.ShapeDtypeStruct((BATCH * 16,), jnp.float32)] * 3,
        scratch_types=[
            pltpu.VMEM((C,), jnp.int32),
            pltpu.VMEM((C, PDIM), jnp.float32),
            pltpu.VMEM((C, PDIM), jnp.float32),
            pltpu.VMEM((C * DIM,), jnp.float32),
            pltpu.VMEM((C * 16,), jnp.float32),
            pltpu.VMEM((C * 16,), jnp.float32),
            pltpu.VMEM((C * 16,), jnp.float32),
            pltpu.VMEM((DIM,), jnp.float32),
            pltpu.SemaphoreType.DMA,
        ],
    )
    def k(cidx_hbm, cemb_hbm, u_hbm, s_hbm, ws_hbm,
          pos_out, neg_out, sent_out,
          cidx_v, vrows, ubuf, sbuf, posb, negb, sentb, ws_v, sem):
        wid = lax.axis_index("s") * NC + lax.axis_index("c")
        pltpu.sync_copy(ws_hbm, ws_v)
        wsv = [ws_v[pl.ds(kk * 16, 16)] for kk in range(NVREG)]

        def chunk_body(c, _):
            base = wid * BPW + c * C
            pltpu.sync_copy(cidx_hbm.at[pl.ds(base, C)], cidx_v)
            cp1 = pltpu.async_copy(cemb_hbm.at[cidx_v], vrows, sem)
            cp2 = pltpu.async_copy(u_hbm.at[pl.ds(base, C)], ubuf, sem)
            cp3 = pltpu.async_copy(s_hbm.at[pl.ds(base * DIM, C * DIM)],
                                   sbuf, sem)
            cp1.wait()
            cp2.wait()
            cp3.wait()

            def b_body(b, _):
                v = [vrows[b, pl.ds(kk * 16, 16)] for kk in range(NVREG)]
                u = [ubuf[b, pl.ds(kk * 16, 16)] for kk in range(NVREG)]
                s = [sbuf[pl.ds(b * DIM + kk * 16, 16)]
                     for kk in range(NVREG)]
                pos = v[0] * u[0]
                sent = v[0] * wsv[0]
                neg = v[0] * s[0]
                for kk in range(1, NVREG):
                    pos = pos + v[kk] * u[kk]
                    sent = sent + v[kk] * wsv[kk]
                    neg = neg + v[kk] * s[kk]
                posb[pl.ds(b * 16, 16)] = pos
                sentb[pl.ds(b * 16, 16)] = sent
                negb[pl.ds(b * 16, 16)] = neg
                return 0

            lax.fori_loop(0, C, b_body, 0)
            pltpu.sync_copy(posb, pos_out.at[pl.ds(base * 16, C * 16)])
            pltpu.sync_copy(negb, neg_out.at[pl.ds(base * 16, C * 16)])
            pltpu.sync_copy(sentb, sent_out.at[pl.ds(base * 16, C * 16)])
            return 0

        lax.fori_loop(0, NCHUNK, chunk_body, 0)

    return k(center_idx, center_emb, u_rows, s_rows, ws)


def _tc_loss(pos_p, neg_p, sent_p):
    def body(pos_ref, neg_ref, sent_ref, out_ref):
        lane = lax.broadcasted_iota(jnp.int32, (128, 8), 0)
        grp = lax.broadcasted_iota(jnp.int32, (128, 8), 1)
        m = (lane // 16 == grp).astype(jnp.float32)
        pos = jnp.dot(pos_ref[...], m, preferred_element_type=jnp.float32)
        neg = jnp.dot(neg_ref[...], m, preferred_element_type=jnp.float32)
        sent = jnp.dot(sent_ref[...], m, preferred_element_type=jnp.float32)

        def log_sigmoid(x):
            return jnp.minimum(x, 0.0) - jnp.log1p(jnp.exp(-jnp.abs(x)))

        pos_val = log_sigmoid(pos)
        neg_val = log_sigmoid(-neg)
        sv = jax.nn.sigmoid(sent + INTERCEPT)
        sent_val = -jnp.abs(sv - 0.5)
        loss = pos_val + sent_val + neg_val
        out_ref[0, 0] = -jnp.sum(loss) / BATCH

    out = pl.pallas_call(
        body,
        out_shape=jax.ShapeDtypeStruct((1, 1), jnp.float32),
        out_specs=pl.BlockSpec(memory_space=pltpu.SMEM),
    )(pos_p, neg_p, sent_p)
    return out[0, 0]


def kernel(center_input, context_output, negative_samples, center_emb,
           context_emb, word_semantics):
    cidx = center_input.astype(jnp.int32)
    uidx = context_output.astype(jnp.int32)
    nidx = negative_samples.astype(jnp.int32).reshape(BATCH * NEG)
    upad = _tc_pad(context_emb.T)
    cpad = _tc_pad(center_emb.T)
    u_rows, s_rows = _sc_context(uidx, nidx, upad)
    pos_p, neg_p, sent_p = _sc_dots(cidx, cpad, u_rows, s_rows,
                                    word_semantics)
    shp = (BATCH * 16 // 128, 128)
    return _tc_loss(pos_p.reshape(shp), neg_p.reshape(shp),
                    sent_p.reshape(shp))
